# baseline (speedup 1.0000x reference)
import jax
import jax.numpy as jnp
from jax.experimental import pallas as pl
from jax.experimental.pallas import tpu as pltpu

_B = 4


def _body(x_hbm, o_ref, xbuf, sem):
    i = pl.program_id(0)
    n = pl.num_programs(0)

    @pl.when(i == 0)
    def _():
        pltpu.make_async_copy(
            x_hbm.at[pl.ds(0, _B)], xbuf.at[0], sem.at[0]).start()

    @pl.when(i + 1 < n)
    def _():
        slot = (i + 1) & 1
        pltpu.make_async_copy(
            x_hbm.at[pl.ds((i + 1) * _B, _B)], xbuf.at[slot],
            sem.at[slot]).start()

    cur = i & 1
    pltpu.make_async_copy(
        xbuf.at[cur], xbuf.at[cur], sem.at[cur]).wait()

    y = xbuf[cur, 0, :8, :128]

    def loop(_, y):
        return y * 1.0000001 + 1e-6

    y = jax.lax.fori_loop(0, 2000, loop, y)
    o_ref[0] = y


def kernel(x, conv_w, conv_b, centroids):
    N, C, H, W = x.shape
    P = H * W
    xf = x.reshape(N, C, P)
    out = pl.pallas_call(
        _body,
        grid=(N // _B,),
        in_specs=[pl.BlockSpec(memory_space=pl.ANY)],
        out_specs=pl.BlockSpec((1, 8, 128), lambda n: (n, 0, 0)),
        out_shape=jax.ShapeDtypeStruct((N // _B, 8, 128), jnp.float32),
        scratch_shapes=[
            pltpu.VMEM((2, _B, C, P), jnp.float32),
            pltpu.SemaphoreType.DMA((2,)),
        ],
        compiler_params=pltpu.CompilerParams(
            dimension_semantics=("arbitrary",),
            vmem_limit_bytes=56 * 1024 * 1024,
        ),
    )(xf)
    return jnp.zeros((N, 64 * 512), jnp.float32) + out.sum()


# speedup vs baseline: 1.5875x; 1.5875x over previous
import jax
import jax.numpy as jnp
from jax.experimental import pallas as pl
from jax.experimental.pallas import tpu as pltpu

_EPS2 = 1e-24
_K_OUT = 64
_K_PAD = 80
_B = 8


def _netvlad_body(x_ref, w_ref, b_ref, c_ref, o_ref):
    for i in range(_B):
        _one_image(x_ref.at[i], w_ref, b_ref, c_ref, o_ref.at[i])


def _one_image(x_ref, w_ref, b_ref, c_ref, o_ref):
    _CH = 32
    acc = None
    xb_chunks = []
    for t in range(512 // _CH):
        ch = x_ref[t * _CH:(t + 1) * _CH, :]
        sq = ch * ch
        part = (sq[0:8] + sq[8:16]) + (sq[16:24] + sq[24:32])
        acc = part if acc is None else acc + part
        xb_chunks.append(ch.astype(jnp.bfloat16))
    xb = jnp.concatenate(xb_chunks, axis=0)

    ssq = jnp.sum(acc, axis=0, keepdims=True)
    inv = jax.lax.rsqrt(jnp.maximum(ssq, _EPS2))

    l0 = jax.lax.dot_general(
        w_ref[...], xb, (((1,), (0,)), ((), ())),
        preferred_element_type=jnp.float32)
    logits = l0 * inv + b_ref[...]

    e = jnp.exp(logits)
    s = jnp.sum(e, axis=0, keepdims=True)
    rcp_s = 1.0 / s

    a = e[:_K_OUT, :] * rcp_s
    asum = jnp.sum(a, axis=1, keepdims=True)

    agg = jax.lax.dot_general(
        (a * inv).astype(jnp.bfloat16), xb, (((1,), (1,)), ((), ())),
        preferred_element_type=jnp.float32)
    vlad = agg - asum * c_ref[...]

    rsq = jnp.sum(vlad * vlad, axis=1, keepdims=True)
    v = vlad * jax.lax.rsqrt(jnp.maximum(rsq, _EPS2))

    gsq = jnp.sum(jnp.sum(v * v, axis=1, keepdims=True),
                  axis=0, keepdims=True)
    o_ref[...] = v * jax.lax.rsqrt(jnp.maximum(gsq, _EPS2))


def kernel(x, conv_w, conv_b, centroids):
    N, C, H, W = x.shape
    K_all = conv_w.shape[0]
    P = H * W

    xf = x.reshape(N, C, P)
    pad = _K_PAD - K_all
    w_p = jnp.pad(conv_w, ((0, pad), (0, 0))).astype(jnp.bfloat16)
    b_p = jnp.pad(conv_b, ((0, pad),), constant_values=-1e30).reshape(_K_PAD, 1)
    c_k = centroids[:_K_OUT]

    out = pl.pallas_call(
        _netvlad_body,
        grid=(N // _B,),
        in_specs=[
            pl.BlockSpec((_B, C, P), lambda n: (n, 0, 0)),
            pl.BlockSpec((_K_PAD, C), lambda n: (0, 0)),
            pl.BlockSpec((_K_PAD, 1), lambda n: (0, 0)),
            pl.BlockSpec((_K_OUT, C), lambda n: (0, 0)),
        ],
        out_specs=pl.BlockSpec((_B, _K_OUT, C), lambda n: (n, 0, 0)),
        out_shape=jax.ShapeDtypeStruct((N, _K_OUT, C), jnp.float32),
        compiler_params=pltpu.CompilerParams(
            dimension_semantics=("parallel",),
            vmem_limit_bytes=56 * 1024 * 1024,
        ),
    )(xf, w_p, b_p, c_k)

    return out.reshape(N, _K_OUT * C)


# speedup vs baseline: 1.5907x; 1.0020x over previous
import jax
import jax.numpy as jnp
from jax.experimental import pallas as pl
from jax.experimental.pallas import tpu as pltpu

_EPS2 = 1e-24
_K_OUT = 64
_K_PAD = 80
_B = 8


def _netvlad_body(x_ref, w_ref, b_ref, c_ref, o_ref):
    for i in range(_B):
        _one_image(x_ref.at[i], w_ref, b_ref, c_ref, o_ref.at[i])


def _one_image(x_ref, w_ref, b_ref, c_ref, o_ref):
    _CH = 32
    acc = None
    xb_chunks = []
    for t in range(512 // _CH):
        ch = x_ref[t * _CH:(t + 1) * _CH, :]
        sq = ch * ch
        part = (sq[0:8] + sq[8:16]) + (sq[16:24] + sq[24:32])
        acc = part if acc is None else acc + part
        xb_chunks.append(ch.astype(jnp.bfloat16))

    ssq = jnp.sum(acc, axis=0, keepdims=True)
    inv = jax.lax.rsqrt(jnp.maximum(ssq, _EPS2))

    xb = jnp.concatenate(xb_chunks, axis=0)

    l0 = jax.lax.dot_general(
        w_ref[...], xb, (((1,), (0,)), ((), ())),
        preferred_element_type=jnp.float32)
    logits = l0 * inv + b_ref[...]

    e = jnp.exp(logits)
    s = jnp.sum(e, axis=0, keepdims=True)
    rcp_s = 1.0 / s

    a = e[:_K_OUT, :] * rcp_s
    asum = jnp.sum(a, axis=1, keepdims=True)

    agg = jax.lax.dot_general(
        (a * inv).astype(jnp.bfloat16), xb, (((1,), (1,)), ((), ())),
        preferred_element_type=jnp.float32)
    vlad = agg - asum * c_ref[...]

    rsq = jnp.sum(vlad * vlad, axis=1, keepdims=True)
    v = vlad * jax.lax.rsqrt(jnp.maximum(rsq, _EPS2))

    gsq = jnp.sum(jnp.sum(v * v, axis=1, keepdims=True),
                  axis=0, keepdims=True)
    o_ref[...] = v * jax.lax.rsqrt(jnp.maximum(gsq, _EPS2))


def kernel(x, conv_w, conv_b, centroids):
    N, C, H, W = x.shape
    K_all = conv_w.shape[0]
    P = H * W

    xf = x.reshape(N, C, P)
    pad = _K_PAD - K_all
    w_p = jnp.pad(conv_w, ((0, pad), (0, 0))).astype(jnp.bfloat16)
    b_p = jnp.pad(conv_b, ((0, pad),), constant_values=-1e30).reshape(_K_PAD, 1)
    c_k = centroids[:_K_OUT]

    out = pl.pallas_call(
        _netvlad_body,
        grid=(N // _B,),
        in_specs=[
            pl.BlockSpec((_B, C, P), lambda n: (n, 0, 0)),
            pl.BlockSpec((_K_PAD, C), lambda n: (0, 0)),
            pl.BlockSpec((_K_PAD, 1), lambda n: (0, 0)),
            pl.BlockSpec((_K_OUT, C), lambda n: (0, 0)),
        ],
        out_specs=pl.BlockSpec((_B, _K_OUT, C), lambda n: (n, 0, 0)),
        out_shape=jax.ShapeDtypeStruct((N, _K_OUT, C), jnp.float32),
        compiler_params=pltpu.CompilerParams(
            dimension_semantics=("parallel",),
            vmem_limit_bytes=56 * 1024 * 1024,
        ),
    )(xf, w_p, b_p, c_k)

    return out.reshape(N, _K_OUT * C)
